# Initial kernel scaffold; baseline (speedup 1.0000x reference)
import jax
import jax.numpy as jnp
from jax.experimental import pallas as pl


def kernel(x, edge_index, Wl0, b0, Wr0, g0, be0, Wl1, b1, Wr1, g1, be1, Wl2, b2, Wr2):
    raise NotImplementedError("write your pallas kernel here")


# speedup vs baseline: 2.9760x; 2.9760x over previous
import functools

import jax
import jax.numpy as jnp
from jax import lax
from jax.experimental import pallas as pl
from jax.experimental.pallas import tpu as pltpu
from jax.experimental.pallas import tpu_sc as plsc

_N = 10000
_E = 320000
_D = 128
_EPS = 1e-5

_NC = 2
_NS = 16
_NW = _NC * _NS
_CH = 128
_CPW = 80
_EPAD = _NW * _CPW * _CH
_NPAD = 10240
_RPT = _NPAD // _NS


def _sc_body(with_cnt, h_hbm, src_hbm, dst_hbm, *refs):
    if with_cnt:
        acc_out, cnt_out = refs[0], refs[1]
        refs = refs[2:]
    else:
        acc_out = refs[0]
        refs = refs[1:]
    src_v, dst_v, rows_v, ones_v, zc_v, acc_sh, cnt_sh, sem = refs

    c = lax.axis_index("c")
    s = lax.axis_index("s")
    wid = s * _NC + c
    base = s * _RPT

    z16 = jnp.zeros((16,), jnp.float32)

    def _zrow(i, carry):
        for k in range(_D // 16):
            rows_v[i, pl.ds(k * 16, 16)] = z16
        return carry

    lax.fori_loop(0, _CH, _zrow, 0)
    for q in range(_RPT // _CH):
        pltpu.sync_copy(rows_v, acc_sh.at[pl.ds(base + q * _CH, _CH)])

    if with_cnt:
        o16 = jnp.ones((16,), jnp.float32)
        for k in range(_CH // 16):
            ones_v[pl.ds(k * 16, 16)] = o16

        def _zc(i, carry):
            zc_v[pl.ds(i * 16, 16)] = z16
            return carry

        lax.fori_loop(0, _RPT // 16, _zc, 0)
        pltpu.sync_copy(zc_v, cnt_sh.at[pl.ds(base, _RPT)])

    cb = wid * _CPW
    pltpu.sync_copy(src_hbm.at[pl.ds(cb, _CPW)], src_v)
    pltpu.sync_copy(dst_hbm.at[pl.ds(cb, _CPW)], dst_v)

    plsc.subcore_barrier()

    def _step(j, carry):
        pltpu.async_copy(h_hbm.at[src_v.at[j]], rows_v, sem).wait()
        pltpu.sync_copy(rows_v, acc_sh.at[dst_v.at[j]], add=True)
        if with_cnt:
            pltpu.sync_copy(ones_v, cnt_sh.at[dst_v.at[j]], add=True)
        return carry

    lax.fori_loop(0, _CPW, _step, 0)

    plsc.subcore_barrier()

    pltpu.sync_copy(acc_sh.at[pl.ds(base, _RPT)],
                    acc_out.at[c, pl.ds(base, _RPT)])
    if with_cnt:
        pltpu.sync_copy(cnt_sh.at[pl.ds(base, _RPT)],
                        cnt_out.at[c, pl.ds(base, _RPT)])


@functools.lru_cache(maxsize=None)
def _make_sc(with_cnt):
    mesh = plsc.VectorSubcoreMesh(core_axis_name="c", subcore_axis_name="s",
                                  num_cores=_NC, num_subcores=_NS)
    out_type = [jax.ShapeDtypeStruct((_NC, _NPAD, _D), jnp.float32)]
    if with_cnt:
        out_type.append(jax.ShapeDtypeStruct((_NC, _NPAD), jnp.float32))
    scratch = [
        pltpu.VMEM((_CPW, _CH), jnp.int32),
        pltpu.VMEM((_CPW, _CH), jnp.int32),
        pltpu.VMEM((_CH, _D), jnp.float32),
        pltpu.VMEM((_CH,), jnp.float32),
        pltpu.VMEM((_RPT,), jnp.float32),
        pltpu.VMEM_SHARED((_NPAD, _D), jnp.float32),
        pltpu.VMEM_SHARED((_NPAD,), jnp.float32),
        pltpu.SemaphoreType.DMA,
    ]
    return pl.kernel(
        functools.partial(_sc_body, with_cnt),
        out_type=out_type,
        mesh=mesh,
        scratch_types=scratch,
    )


def _tc_body(final, x_ref, a_ref, c_ref, wl_ref, b_ref, wr_ref, g_ref,
             be_ref, o_ref):
    a = a_ref[0] + a_ref[1]
    cnt = c_ref[:, 0:1] + c_ref[:, 1:2]
    mean = a / jnp.maximum(cnt, 1.0)
    x = x_ref[...]
    o = lax.dot_general(mean, wl_ref[...], (((1,), (1,)), ((), ())),
                        preferred_element_type=jnp.float32)
    o = o + b_ref[...]
    o = o + lax.dot_general(x, wr_ref[...], (((1,), (1,)), ((), ())),
                            preferred_element_type=jnp.float32)
    if not final:
        scale = 1.0 / (1.0 + _EPS) ** 0.5
        o = o * (g_ref[...] * scale) + be_ref[...]
        o = jnp.maximum(o, 0.0) + x
    o_ref[...] = o


@functools.lru_cache(maxsize=None)
def _make_tc(final):
    grid = (_NPAD // _RPT,)
    in_specs = [
        pl.BlockSpec((_RPT, _D), lambda i: (i, 0)),
        pl.BlockSpec((_NC, _RPT, _D), lambda i: (0, i, 0)),
        pl.BlockSpec((_RPT, _NC), lambda i: (i, 0)),
        pl.BlockSpec((_D, _D), lambda i: (0, 0)),
        pl.BlockSpec((1, _D), lambda i: (0, 0)),
        pl.BlockSpec((_D, _D), lambda i: (0, 0)),
        pl.BlockSpec((1, _D), lambda i: (0, 0)),
        pl.BlockSpec((1, _D), lambda i: (0, 0)),
    ]
    return pl.pallas_call(
        functools.partial(_tc_body, final),
        grid=grid,
        in_specs=in_specs,
        out_specs=pl.BlockSpec((_RPT, _D), lambda i: (i, 0)),
        out_shape=jax.ShapeDtypeStruct((_NPAD, _D), jnp.float32),
    )


def kernel(x, edge_index, Wl0, b0, Wr0, g0, be0, Wl1, b1, Wr1, g1, be1,
           Wl2, b2, Wr2):
    src = edge_index[0]
    dst = edge_index[1]
    srcp = jnp.concatenate(
        [src, jnp.zeros((_EPAD - _E,), jnp.int32)]).reshape(_NW * _CPW, _CH)
    dstp = jnp.concatenate(
        [dst, jnp.full((_EPAD - _E,), _N, jnp.int32)]).reshape(_NW * _CPW, _CH)
    xp = jnp.concatenate([x, jnp.zeros((_NPAD - _N, _D), jnp.float32)])

    sc0 = _make_sc(True)
    sc = _make_sc(False)
    tc = _make_tc(False)
    tc_final = _make_tc(True)

    A, cnt = sc0(xp, srcp, dstp)
    cntT = cnt.T
    h = tc(xp, A, cntT, Wl0, b0.reshape(1, _D), Wr0, g0.reshape(1, _D),
           be0.reshape(1, _D))
    (A,) = sc(h, srcp, dstp)
    h = tc(h, A, cntT, Wl1, b1.reshape(1, _D), Wr1, g1.reshape(1, _D),
           be1.reshape(1, _D))
    (A,) = sc(h, srcp, dstp)
    h = tc_final(h, A, cntT, Wl2, b2.reshape(1, _D), Wr2, g2_dummy(),
                 be_dummy())
    return h[:_N]


def g2_dummy():
    return jnp.ones((1, _D), jnp.float32)


def be_dummy():
    return jnp.zeros((1, _D), jnp.float32)
